# baseline (speedup 1.0000x reference)
import functools

import jax
import jax.numpy as jnp
from jax import lax
from jax.experimental import pallas as pl
from jax.experimental.pallas import tpu as pltpu
from jax.experimental.pallas import tpu_sc as plsc

NC, NS = 2, 16
NW = NC * NS
CHUNK = 128
LANES = 16
KBUF = 4
LEAD = 2


def kernel(words, tags, word_table, pos_table):
    B, L = words.shape
    D = word_table.shape[1]
    T = pos_table.shape[0]
    N = B * L
    n_per_w = N // NW
    n_chunks = n_per_w // CHUNK
    n_groups = n_chunks // KBUF

    words_flat = words.reshape(N).astype(jnp.int32)
    tags_flat = tags.reshape(N).astype(jnp.int32)
    wtab128 = jnp.concatenate([word_table, word_table], axis=1)

    mesh = plsc.VectorSubcoreMesh(
        core_axis_name="c", subcore_axis_name="s",
        num_cores=NC, num_subcores=NS)

    @functools.partial(
        pl.kernel,
        out_type=jax.ShapeDtypeStruct((N, 2 * D), jnp.float32),
        mesh=mesh,
        compiler_params=pltpu.CompilerParams(needs_layout_passes=False),
        scratch_types=[
            pltpu.VMEM((n_per_w,), jnp.int32),
            pltpu.VMEM((n_per_w,), jnp.int32),
            pltpu.VMEM((T, D), jnp.float32),
            [pltpu.VMEM((CHUNK, 2 * D), jnp.float32)] * KBUF,
            [pltpu.SemaphoreType.DMA] * KBUF,
            [pltpu.SemaphoreType.DMA] * KBUF,
        ],
    )
    def run(words_hbm, tags_hbm, wtab_hbm, ptab_hbm, out_hbm,
            widx, tidx, ptab, combs, gsems, wsems):
        wid = lax.axis_index("s") * NC + lax.axis_index("c")
        base0 = wid * n_per_w
        pltpu.sync_copy(words_hbm.at[pl.ds(base0, n_per_w)], widx)
        pltpu.sync_copy(tags_hbm.at[pl.ds(base0, n_per_w)], tidx)
        pltpu.sync_copy(ptab_hbm, ptab)

        def issue_gather(c, b):
            pltpu.async_copy(wtab_hbm.at[widx.at[pl.ds(c * CHUNK, CHUNK)]],
                             combs[b], gsems[b])

        def wait_gather(b):
            pltpu.make_async_copy(wtab_hbm.at[widx.at[pl.ds(0, CHUNK)]],
                                  combs[b], gsems[b]).wait()

        def issue_wb(c, b):
            pltpu.async_copy(combs[b],
                             out_hbm.at[pl.ds(base0 + c * CHUNK, CHUNK), :],
                             wsems[b])

        def wait_wb(b):
            pltpu.make_async_copy(combs[b],
                                  out_hbm.at[pl.ds(base0, CHUNK), :],
                                  wsems[b]).wait()

        for c in range(LEAD):
            issue_gather(c, c % KBUF)

        def group(k, carry):
            for b in range(KBUF):
                c = k * KBUF + b
                bb = (b + LEAD) % KBUF
                if b < LEAD:
                    @pl.when(k > 0)
                    def _():
                        wait_wb(bb)

                    issue_gather(c + LEAD, bb)
                else:
                    wait_wb(bb)

                    @pl.when(k < n_groups - 1)
                    def _():
                        issue_gather(c + LEAD, bb)

                wait_gather(b)
                off = c * CHUNK

                @plsc.parallel_loop(0, CHUNK // LANES)
                def fill(g):
                    tvec = tidx[pl.ds(off + g * LANES, LANES)]
                    for rr in range(LANES):
                        t = tvec[rr]
                        r = g * LANES + rr
                        for j in range(D // LANES):
                            combs[b][r, pl.ds(D + LANES * j, LANES)] = (
                                ptab[t, pl.ds(LANES * j, LANES)])

                issue_wb(c, b)
            return carry

        lax.fori_loop(0, n_groups, group, 0)
        for i in range(LEAD):
            wait_wb((n_chunks - LEAD + i) % KBUF)

    out = run(words_flat, tags_flat, wtab128, pos_table)
    return out.reshape(B, L, 2 * D)


# speedup vs baseline: 1.4837x; 1.4837x over previous
import functools

import jax
import jax.numpy as jnp
from jax import lax
from jax.experimental import pallas as pl
from jax.experimental.pallas import tpu as pltpu
from jax.experimental.pallas import tpu_sc as plsc

NC, NS = 2, 16
NW = NC * NS
CHUNK = 128
LANES = 16
KBUF = 4
LEAD = 2


def _widen_table(word_table):
    V, D = word_table.shape
    BQ = 4096
    VP = -(-V // BQ) * BQ

    def body(wt_t_ref, out_ref):
        out_ref[:, 0:D] = wt_t_ref[...].T

    return pl.pallas_call(
        body,
        grid=(VP // BQ,),
        in_specs=[pl.BlockSpec((D, BQ), lambda q: (0, q))],
        out_specs=pl.BlockSpec((BQ, 2 * D), lambda q: (q, 0)),
        out_shape=jax.ShapeDtypeStruct((VP, 2 * D), jnp.float32),
    )(word_table.T)


def kernel(words, tags, word_table, pos_table):
    B, L = words.shape
    D = word_table.shape[1]
    T = pos_table.shape[0]
    N = B * L
    n_per_w = N // NW
    n_chunks = n_per_w // CHUNK
    n_groups = n_chunks // KBUF

    words_flat = words.reshape(N).astype(jnp.int32)
    tags_flat = tags.reshape(N).astype(jnp.int32)
    wtab128 = _widen_table(word_table)

    mesh = plsc.VectorSubcoreMesh(
        core_axis_name="c", subcore_axis_name="s",
        num_cores=NC, num_subcores=NS)

    @functools.partial(
        pl.kernel,
        out_type=jax.ShapeDtypeStruct((N, 2 * D), jnp.float32),
        mesh=mesh,
        compiler_params=pltpu.CompilerParams(needs_layout_passes=False),
        scratch_types=[
            pltpu.VMEM((n_per_w,), jnp.int32),
            pltpu.VMEM((n_per_w,), jnp.int32),
            pltpu.VMEM((T, D), jnp.float32),
            [pltpu.VMEM((CHUNK, 2 * D), jnp.float32)] * KBUF,
            [pltpu.SemaphoreType.DMA] * KBUF,
            [pltpu.SemaphoreType.DMA] * KBUF,
        ],
    )
    def run(words_hbm, tags_hbm, wtab_hbm, ptab_hbm, out_hbm,
            widx, tidx, ptab, combs, gsems, wsems):
        wid = lax.axis_index("s") * NC + lax.axis_index("c")
        base0 = wid * n_per_w
        pltpu.sync_copy(words_hbm.at[pl.ds(base0, n_per_w)], widx)
        pltpu.sync_copy(tags_hbm.at[pl.ds(base0, n_per_w)], tidx)
        pltpu.sync_copy(ptab_hbm, ptab)

        def issue_gather(c, b):
            pltpu.async_copy(wtab_hbm.at[widx.at[pl.ds(c * CHUNK, CHUNK)]],
                             combs[b], gsems[b])

        def wait_gather(b):
            pltpu.make_async_copy(wtab_hbm.at[widx.at[pl.ds(0, CHUNK)]],
                                  combs[b], gsems[b]).wait()

        def issue_wb(c, b):
            pltpu.async_copy(combs[b],
                             out_hbm.at[pl.ds(base0 + c * CHUNK, CHUNK), :],
                             wsems[b])

        def wait_wb(b):
            pltpu.make_async_copy(combs[b],
                                  out_hbm.at[pl.ds(base0, CHUNK), :],
                                  wsems[b]).wait()

        for c in range(LEAD):
            issue_gather(c, c % KBUF)

        def group(k, carry):
            for b in range(KBUF):
                c = k * KBUF + b
                bb = (b + LEAD) % KBUF
                if b < LEAD:
                    @pl.when(k > 0)
                    def _():
                        wait_wb(bb)

                    issue_gather(c + LEAD, bb)
                else:
                    wait_wb(bb)

                    @pl.when(k < n_groups - 1)
                    def _():
                        issue_gather(c + LEAD, bb)

                wait_gather(b)
                off = c * CHUNK

                @plsc.parallel_loop(0, CHUNK // LANES)
                def fill(g):
                    tvec = tidx[pl.ds(off + g * LANES, LANES)]
                    for rr in range(LANES):
                        t = tvec[rr]
                        r = g * LANES + rr
                        for j in range(D // LANES):
                            combs[b][r, pl.ds(D + LANES * j, LANES)] = (
                                ptab[t, pl.ds(LANES * j, LANES)])

                issue_wb(c, b)
            return carry

        lax.fori_loop(0, n_groups, group, 0)
        for i in range(LEAD):
            wait_wb((n_chunks - LEAD + i) % KBUF)

    out = run(words_flat, tags_flat, wtab128, pos_table)
    return out.reshape(B, L, 2 * D)


# speedup vs baseline: 1.5342x; 1.0341x over previous
import functools

import jax
import jax.numpy as jnp
from jax import lax
from jax.experimental import pallas as pl
from jax.experimental.pallas import tpu as pltpu
from jax.experimental.pallas import tpu_sc as plsc

NC, NS = 2, 16
NW = NC * NS
CHUNK = 128
LANES = 16
KBUF = 4
LEAD = 2
BQ = 4096


def _pack_table(word_table, vp2):
    V, D = word_table.shape
    nq = vp2 // BQ
    vlast = -(-V // BQ) - 1

    def body(lo_ref, hi_ref, out_ref):
        out_ref[:, 0:D] = lo_ref[...].T
        out_ref[:, D:2 * D] = hi_ref[...].T

    return pl.pallas_call(
        body,
        grid=(nq,),
        in_specs=[pl.BlockSpec((D, BQ), lambda q: (0, q)),
                  pl.BlockSpec((D, BQ),
                               lambda q, _n=nq, _l=vlast:
                               (0, jnp.minimum(q + _n, _l)))],
        out_specs=pl.BlockSpec((BQ, 2 * D), lambda q: (q, 0)),
        out_shape=jax.ShapeDtypeStruct((vp2, 2 * D), jnp.float32),
    )(word_table.T, word_table.T)


def kernel(words, tags, word_table, pos_table):
    B, L = words.shape
    D = word_table.shape[1]
    T = pos_table.shape[0]
    N = B * L
    n_per_w = N // NW
    n_chunks = n_per_w // CHUNK
    n_groups = n_chunks // KBUF

    V = word_table.shape[0]
    vp2 = -(-((V + 1) // 2) // BQ) * BQ

    words_flat = words.reshape(N).astype(jnp.int32)
    tags_flat = tags.reshape(N).astype(jnp.int32)
    wpacked = _pack_table(word_table, vp2)

    mesh = plsc.VectorSubcoreMesh(
        core_axis_name="c", subcore_axis_name="s",
        num_cores=NC, num_subcores=NS)

    @functools.partial(
        pl.kernel,
        out_type=jax.ShapeDtypeStruct((N, 2 * D), jnp.float32),
        mesh=mesh,
        compiler_params=pltpu.CompilerParams(needs_layout_passes=False),
        scratch_types=[
            pltpu.VMEM((n_per_w,), jnp.int32),
            pltpu.VMEM((n_per_w,), jnp.int32),
            pltpu.VMEM((T, D), jnp.float32),
            [pltpu.VMEM((CHUNK, 2 * D), jnp.float32)] * KBUF,
            [pltpu.SemaphoreType.DMA] * KBUF,
            [pltpu.SemaphoreType.DMA] * KBUF,
        ],
    )
    def run(words_hbm, tags_hbm, wtab_hbm, ptab_hbm, out_hbm,
            widx, tidx, ptab, combs, gsems, wsems):
        wid = lax.axis_index("s") * NC + lax.axis_index("c")
        base0 = wid * n_per_w
        pltpu.sync_copy(words_hbm.at[pl.ds(base0, n_per_w)], widx)
        pltpu.sync_copy(tags_hbm.at[pl.ds(base0, n_per_w)], tidx)
        pltpu.sync_copy(ptab_hbm, ptab)

        @plsc.parallel_loop(0, n_per_w // LANES, unroll=4)
        def split(g):
            sl = pl.ds(g * LANES, LANES)
            w = widx[sl]
            hi = w >= vp2
            widx[sl] = jnp.where(hi, w - vp2, w)
            tidx[sl] = jnp.where(hi, tidx[sl] + 256, tidx[sl])

        def issue_gather(c, b):
            pltpu.async_copy(wtab_hbm.at[widx.at[pl.ds(c * CHUNK, CHUNK)]],
                             combs[b], gsems[b])

        def wait_gather(b):
            pltpu.make_async_copy(wtab_hbm.at[widx.at[pl.ds(0, CHUNK)]],
                                  combs[b], gsems[b]).wait()

        def issue_wb(c, b):
            pltpu.async_copy(combs[b],
                             out_hbm.at[pl.ds(base0 + c * CHUNK, CHUNK), :],
                             wsems[b])

        def wait_wb(b):
            pltpu.make_async_copy(combs[b],
                                  out_hbm.at[pl.ds(base0, CHUNK), :],
                                  wsems[b]).wait()

        for c in range(LEAD):
            issue_gather(c, c % KBUF)

        def group(k, carry):
            for b in range(KBUF):
                c = k * KBUF + b
                bb = (b + LEAD) % KBUF
                if b < LEAD:
                    @pl.when(k > 0)
                    def _():
                        wait_wb(bb)

                    issue_gather(c + LEAD, bb)
                else:
                    wait_wb(bb)

                    @pl.when(k < n_groups - 1)
                    def _():
                        issue_gather(c + LEAD, bb)

                wait_gather(b)
                off0 = c * CHUNK

                @plsc.parallel_loop(0, CHUNK // LANES)
                def fill(g):
                    tvec = tidx[pl.ds(off0 + g * LANES, LANES)]
                    for rr in range(LANES):
                        tp = tvec[rr]
                        t = tp & 255
                        off = (tp >> 2) & D
                        r = g * LANES + rr
                        wv = [combs[b][r, pl.ds(off + LANES * j, LANES)]
                              for j in range(D // LANES)]
                        for j in range(D // LANES):
                            combs[b][r, pl.ds(LANES * j, LANES)] = wv[j]
                        for j in range(D // LANES):
                            combs[b][r, pl.ds(D + LANES * j, LANES)] = (
                                ptab[t, pl.ds(LANES * j, LANES)])

                issue_wb(c, b)
            return carry

        lax.fori_loop(0, n_groups, group, 0)
        for i in range(LEAD):
            wait_wb((n_chunks - LEAD + i) % KBUF)

    out = run(words_flat, tags_flat, wpacked, pos_table)
    return out.reshape(B, L, 2 * D)
